# Initial kernel scaffold; baseline (speedup 1.0000x reference)
import jax
import jax.numpy as jnp
from jax.experimental import pallas as pl


def kernel(pos_s, pos_e, pe_table, W, b):
    raise NotImplementedError("write your pallas kernel here")


# speedup vs baseline: 14.6953x; 14.6953x over previous
import functools

import jax
import jax.numpy as jnp
from jax import lax
from jax.experimental import pallas as pl
from jax.experimental.pallas import tpu as pltpu
from jax.experimental.pallas import tpu_sc as plsc

_B = 2
_L = 512
_H = 128
_NT = 4
_TR = 2 * _L
_JB = 128
_NW = 32


def _fuse_tables_body(pe_ref, w_ref, b_ref, out_ref):
    k = pl.program_id(0)
    acc = jnp.dot(pe_ref[...], w_ref[...], preferred_element_type=jnp.float32)
    out_ref[...] = acc + jnp.where(k == 0, b_ref[...], 0.0)


def _build_tables(pe_table, w, b):
    return pl.pallas_call(
        _fuse_tables_body,
        grid=(_NT,),
        in_specs=[
            pl.BlockSpec((_TR, _H), lambda k: (0, 0)),
            pl.BlockSpec((_H, _H), lambda k: (k, 0)),
            pl.BlockSpec((1, _H), lambda k: (0, 0)),
        ],
        out_specs=pl.BlockSpec((_TR, _H), lambda k: (k, 0)),
        out_shape=jax.ShapeDtypeStruct((_NT * _TR, _H), jnp.float32),
    )(pe_table, w, b.reshape(1, _H))


def _sc_fuse(tcat, pos_sf, pos_ef, asp, esp, out,
             pos_sv, pos_ev, a_sv, a_ev, i0, i1, i2, i3, g0, g1, g2, g3,
             obuf, sem):
    wid = lax.axis_index("s") * 2 + lax.axis_index("c")
    pltpu.sync_copy(pos_sf, pos_sv)
    pltpu.sync_copy(pos_ef, pos_ev)
    rows_per_w = (_B * _L) // _NW

    def row_body(n, _):
        r = wid * rows_per_w + n
        b_idx = r // _L
        base_col = b_idx * _L
        pltpu.sync_copy(asp.at[r], a_sv)
        pltpu.sync_copy(esp.at[r], a_ev)
        a_s = a_sv[...]
        a_e = a_ev[...]

        def jblock(jb, _):
            col0 = base_col + jb * _JB
            for t in range(_JB // 16):
                sl = pl.ds(t * 16, 16)
                c_s = pos_sv[pl.ds(col0 + t * 16, 16)]
                c_e = pos_ev[pl.ds(col0 + t * 16, 16)]
                i0[sl] = a_s - c_s + _L
                i1[sl] = a_s - c_e + (_TR + _L)
                i2[sl] = a_e - c_s + (2 * _TR + _L)
                i3[sl] = a_e - c_e + (3 * _TR + _L)
            cp0 = pltpu.async_copy(tcat.at[i0], g0, sem)
            cp1 = pltpu.async_copy(tcat.at[i1], g1, sem)
            cp2 = pltpu.async_copy(tcat.at[i2], g2, sem)
            cp3 = pltpu.async_copy(tcat.at[i3], g3, sem)
            cp0.wait()
            cp1.wait()
            cp2.wait()
            cp3.wait()

            def jrow(j, _):
                for h in range(_H // 16):
                    sl = pl.ds(h * 16, 16)
                    acc = g0[j, sl] + g1[j, sl] + g2[j, sl] + g3[j, sl]
                    obuf[j, sl] = jnp.maximum(acc, 0.0)
                return 0

            lax.fori_loop(0, _JB, jrow, 0)
            pltpu.sync_copy(obuf, out.at[pl.ds(r * _L + jb * _JB, _JB)])
            return 0

        lax.fori_loop(0, _L // _JB, jblock, 0)
        return 0

    lax.fori_loop(0, rows_per_w, row_body, 0)


_sc_call = pl.kernel(
    _sc_fuse,
    mesh=plsc.VectorSubcoreMesh(core_axis_name="c", subcore_axis_name="s"),
    out_type=jax.ShapeDtypeStruct((_B * _L * _L, _H), jnp.float32),
    scratch_types=[
        pltpu.VMEM((_B * _L,), jnp.int32),
        pltpu.VMEM((_B * _L,), jnp.int32),
        pltpu.VMEM((16,), jnp.int32),
        pltpu.VMEM((16,), jnp.int32),
        pltpu.VMEM((_JB,), jnp.int32),
        pltpu.VMEM((_JB,), jnp.int32),
        pltpu.VMEM((_JB,), jnp.int32),
        pltpu.VMEM((_JB,), jnp.int32),
        pltpu.VMEM((_JB, _H), jnp.float32),
        pltpu.VMEM((_JB, _H), jnp.float32),
        pltpu.VMEM((_JB, _H), jnp.float32),
        pltpu.VMEM((_JB, _H), jnp.float32),
        pltpu.VMEM((_JB, _H), jnp.float32),
        pltpu.SemaphoreType.DMA,
    ],
)


def kernel(pos_s, pos_e, pe_table, W, b):
    tcat = _build_tables(pe_table, W, b)
    psf = pos_s.reshape(-1).astype(jnp.int32)
    pef = pos_e.reshape(-1).astype(jnp.int32)
    asp = jnp.broadcast_to(psf[:, None], (_B * _L, 16)) + 0
    esp = jnp.broadcast_to(pef[:, None], (_B * _L, 16)) + 0
    out = _sc_call(tcat, psf, pef, asp, esp)
    return out.reshape(_B, _L, _L, _H)


# speedup vs baseline: 16.7003x; 1.1364x over previous
import functools

import jax
import jax.numpy as jnp
from jax import lax
from jax.experimental import pallas as pl
from jax.experimental.pallas import tpu as pltpu
from jax.experimental.pallas import tpu_sc as plsc

_B = 2
_L = 512
_H = 128
_NT = 4
_TR = 2 * _L
_JB = 128
_NW = 32


def _fuse_tables_body(pe_ref, w_ref, b_ref, out_ref):
    k = pl.program_id(0)
    acc = jnp.dot(pe_ref[...], w_ref[...], preferred_element_type=jnp.float32)
    out_ref[...] = acc + jnp.where(k == 0, b_ref[...], 0.0)


def _build_tables(pe_table, w, b):
    return pl.pallas_call(
        _fuse_tables_body,
        grid=(_NT,),
        in_specs=[
            pl.BlockSpec((_TR, _H), lambda k: (0, 0)),
            pl.BlockSpec((_H, _H), lambda k: (k, 0)),
            pl.BlockSpec((1, _H), lambda k: (0, 0)),
        ],
        out_specs=pl.BlockSpec((_TR, _H), lambda k: (k, 0)),
        out_shape=jax.ShapeDtypeStruct((_NT * _TR, _H), jnp.float32),
    )(pe_table, w, b.reshape(1, _H))


def _sc_fuse(tcat, pos_sf, pos_ef, asp, esp, out,
             pos_sv, pos_ev, a_sv, a_ev, i0, i1, i2, i3, g0, g1, g2, g3,
             obuf, sem):
    wid = lax.axis_index("s") * 2 + lax.axis_index("c")
    pltpu.sync_copy(pos_sf, pos_sv)
    pltpu.sync_copy(pos_ef, pos_ev)
    rows_per_w = (_B * _L) // _NW

    def row_body(n, _):
        r = wid * rows_per_w + n
        b_idx = r // _L
        base_col = b_idx * _L
        pltpu.sync_copy(asp.at[r], a_sv)
        pltpu.sync_copy(esp.at[r], a_ev)
        a_s = a_sv[...]
        a_e = a_ev[...]

        def jblock(jb, _):
            col0 = base_col + jb * _JB
            for t in range(_JB // 16):
                sl = pl.ds(t * 16, 16)
                c_s = pos_sv[pl.ds(col0 + t * 16, 16)]
                c_e = pos_ev[pl.ds(col0 + t * 16, 16)]
                i0[sl] = a_s - c_s + _L
                i1[sl] = a_s - c_e + (_TR + _L)
                i2[sl] = a_e - c_s + (2 * _TR + _L)
                i3[sl] = a_e - c_e + (3 * _TR + _L)
            cp0 = pltpu.async_copy(tcat.at[i0], g0, sem)
            cp0.wait()
            cp1 = pltpu.async_copy(tcat.at[i1], g0, sem, add=True)
            cp2 = pltpu.async_copy(tcat.at[i2], g0, sem, add=True)
            cp3 = pltpu.async_copy(tcat.at[i3], g0, sem, add=True)
            cp1.wait()
            cp2.wait()
            cp3.wait()

            def jrow(j, _):
                for h in range(_H // 16):
                    sl = pl.ds(h * 16, 16)
                    obuf[j, sl] = jnp.maximum(g0[j, sl], 0.0)
                return 0

            lax.fori_loop(0, _JB, jrow, 0)
            pltpu.sync_copy(obuf, out.at[pl.ds(r * _L + jb * _JB, _JB)])
            return 0

        lax.fori_loop(0, _L // _JB, jblock, 0)
        return 0

    lax.fori_loop(0, rows_per_w, row_body, 0)


_sc_call = pl.kernel(
    _sc_fuse,
    mesh=plsc.VectorSubcoreMesh(core_axis_name="c", subcore_axis_name="s"),
    out_type=jax.ShapeDtypeStruct((_B * _L * _L, _H), jnp.float32),
    scratch_types=[
        pltpu.VMEM((_B * _L,), jnp.int32),
        pltpu.VMEM((_B * _L,), jnp.int32),
        pltpu.VMEM((16,), jnp.int32),
        pltpu.VMEM((16,), jnp.int32),
        pltpu.VMEM((_JB,), jnp.int32),
        pltpu.VMEM((_JB,), jnp.int32),
        pltpu.VMEM((_JB,), jnp.int32),
        pltpu.VMEM((_JB,), jnp.int32),
        pltpu.VMEM((_JB, _H), jnp.float32),
        pltpu.VMEM((_JB, _H), jnp.float32),
        pltpu.VMEM((_JB, _H), jnp.float32),
        pltpu.VMEM((_JB, _H), jnp.float32),
        pltpu.VMEM((_JB, _H), jnp.float32),
        pltpu.SemaphoreType.DMA,
    ],
)


def kernel(pos_s, pos_e, pe_table, W, b):
    tcat = _build_tables(pe_table, W, b)
    psf = pos_s.reshape(-1).astype(jnp.int32)
    pef = pos_e.reshape(-1).astype(jnp.int32)
    asp = jnp.broadcast_to(psf[:, None], (_B * _L, 16)) + 0
    esp = jnp.broadcast_to(pef[:, None], (_B * _L, 16)) + 0
    out = _sc_call(tcat, psf, pef, asp, esp)
    return out.reshape(_B, _L, _L, _H)


# speedup vs baseline: 20.8210x; 1.2467x over previous
import functools

import jax
import jax.numpy as jnp
from jax import lax
from jax.experimental import pallas as pl
from jax.experimental.pallas import tpu as pltpu
from jax.experimental.pallas import tpu_sc as plsc

_B = 2
_L = 512
_H = 128
_NT = 4
_TR = 2 * _L
_JB = 128
_NW = 32


def _fuse_tables_body(pe_ref, w_ref, b_ref, out_ref):
    k = pl.program_id(0)
    acc = jnp.dot(pe_ref[...], w_ref[...], preferred_element_type=jnp.float32)
    out_ref[...] = acc + jnp.where(k == 0, b_ref[...], 0.0)


def _build_tables(pe_table, w, b):
    return pl.pallas_call(
        _fuse_tables_body,
        grid=(_NT,),
        in_specs=[
            pl.BlockSpec((_TR, _H), lambda k: (0, 0)),
            pl.BlockSpec((_H, _H), lambda k: (k, 0)),
            pl.BlockSpec((1, _H), lambda k: (0, 0)),
        ],
        out_specs=pl.BlockSpec((_TR, _H), lambda k: (k, 0)),
        out_shape=jax.ShapeDtypeStruct((_NT * _TR, _H), jnp.float32),
    )(pe_table, w, b.reshape(1, _H))


_JPB = _L // _JB
_STEPS = ((_B * _L) // _NW) * _JPB


def _sc_fuse(tcat, pos_sf, pos_ef, asp, esp, out,
             pos_sv, pos_ev, aspv, espv, ix0, ix1, g0, g1,
             gsem0, gsem1, osem0, osem1):
    wid = lax.axis_index("s") * 2 + lax.axis_index("c")
    pltpu.sync_copy(pos_sf, pos_sv)
    pltpu.sync_copy(pos_ef, pos_ev)
    pltpu.sync_copy(asp, aspv)
    pltpu.sync_copy(esp, espv)
    rows_per_w = (_B * _L) // _NW
    r0 = wid * rows_per_w

    def build_idx(s, ix):
        r = r0 + s // _JPB
        jb = s % _JPB
        col0 = (r // _L) * _L + jb * _JB
        a_s = aspv[pl.ds(r * 16, 16)]
        a_e = espv[pl.ds(r * 16, 16)]
        for t in range(_JB // 16):
            sl = pl.ds(t * 16, 16)
            c_s = pos_sv[pl.ds(col0 + t * 16, 16)]
            c_e = pos_ev[pl.ds(col0 + t * 16, 16)]
            ix[0, sl] = a_s - c_s + _L
            ix[1, sl] = a_s - c_e + (_TR + _L)
            ix[2, sl] = a_e - c_s + (2 * _TR + _L)
            ix[3, sl] = a_e - c_e + (3 * _TR + _L)

    def out_slice(s):
        r = r0 + s // _JPB
        jb = s % _JPB
        return out.at[pl.ds(r * _L + jb * _JB, _JB)]

    build_idx(0, ix0)
    pltpu.async_copy(tcat.at[ix0.at[0]], g0, gsem0)

    def step(s, _):
        par = s % 2

        def run(ix, g, gsem, osem, oth_ix, oth_g, oth_gsem, oth_osem):
            pltpu.make_async_copy(tcat.at[ix.at[0]], g, gsem).wait()
            a1 = pltpu.async_copy(tcat.at[ix.at[1]], g, gsem, add=True)
            a2 = pltpu.async_copy(tcat.at[ix.at[2]], g, gsem, add=True)
            a3 = pltpu.async_copy(tcat.at[ix.at[3]], g, gsem, add=True)

            @pl.when(s + 1 < _STEPS)
            def _prefetch():
                build_idx(s + 1, oth_ix)

                @pl.when(s >= 1)
                def _free():
                    pltpu.make_async_copy(oth_g, out_slice(s - 1),
                                          oth_osem).wait()

                pltpu.async_copy(tcat.at[oth_ix.at[0]], oth_g, oth_gsem)

            a1.wait()
            a2.wait()
            a3.wait()

            def jrow(j4, _):
                for u in range(4):
                    j = j4 * 4 + u
                    for h in range(_H // 16):
                        sl = pl.ds(h * 16, 16)
                        g[j, sl] = jnp.maximum(g[j, sl], 0.0)
                return 0

            lax.fori_loop(0, _JB // 4, jrow, 0)
            pltpu.async_copy(g, out_slice(s), osem)

        @pl.when(par == 0)
        def _even():
            run(ix0, g0, gsem0, osem0, ix1, g1, gsem1, osem1)

        @pl.when(par == 1)
        def _odd():
            run(ix1, g1, gsem1, osem1, ix0, g0, gsem0, osem0)

        return 0

    lax.fori_loop(0, _STEPS, step, 0)
    pltpu.make_async_copy(g0, out_slice(_STEPS - 2), osem0).wait()
    pltpu.make_async_copy(g1, out_slice(_STEPS - 1), osem1).wait()


_sc_call = pl.kernel(
    _sc_fuse,
    mesh=plsc.VectorSubcoreMesh(core_axis_name="c", subcore_axis_name="s"),
    out_type=jax.ShapeDtypeStruct((_B * _L * _L, _H), jnp.float32),
    scratch_types=[
        pltpu.VMEM((_B * _L,), jnp.int32),
        pltpu.VMEM((_B * _L,), jnp.int32),
        pltpu.VMEM((_B * _L * 16,), jnp.int32),
        pltpu.VMEM((_B * _L * 16,), jnp.int32),
        pltpu.VMEM((_NT, _JB), jnp.int32),
        pltpu.VMEM((_NT, _JB), jnp.int32),
        pltpu.VMEM((_JB, _H), jnp.float32),
        pltpu.VMEM((_JB, _H), jnp.float32),
        pltpu.SemaphoreType.DMA,
        pltpu.SemaphoreType.DMA,
        pltpu.SemaphoreType.DMA,
        pltpu.SemaphoreType.DMA,
    ],
)


def kernel(pos_s, pos_e, pe_table, W, b):
    tcat = _build_tables(pe_table, W, b)
    psf = pos_s.reshape(-1).astype(jnp.int32)
    pef = pos_e.reshape(-1).astype(jnp.int32)
    asp = jnp.broadcast_to(psf[:, None], (_B * _L, 16)).reshape(-1)
    esp = jnp.broadcast_to(pef[:, None], (_B * _L, 16)).reshape(-1)
    out = _sc_call(tcat, psf, pef, asp, esp)
    return out.reshape(_B, _L, _L, _H)


# speedup vs baseline: 21.0129x; 1.0092x over previous
import functools

import jax
import jax.numpy as jnp
from jax import lax
from jax.experimental import pallas as pl
from jax.experimental.pallas import tpu as pltpu
from jax.experimental.pallas import tpu_sc as plsc

_B = 2
_L = 512
_H = 128
_NT = 4
_TR = 2 * _L
_JB = 128
_NW = 32


def _fuse_tables_body(pe_ref, w_ref, b_ref, out_ref):
    k = pl.program_id(0)
    acc = jnp.dot(pe_ref[...], w_ref[...], preferred_element_type=jnp.float32)
    out_ref[...] = acc + jnp.where(k == 0, b_ref[...], 0.0)


def _build_tables(pe_table, w, b):
    return pl.pallas_call(
        _fuse_tables_body,
        grid=(_NT,),
        in_specs=[
            pl.BlockSpec((_TR, _H), lambda k: (0, 0)),
            pl.BlockSpec((_H, _H), lambda k: (k, 0)),
            pl.BlockSpec((1, _H), lambda k: (0, 0)),
        ],
        out_specs=pl.BlockSpec((_TR, _H), lambda k: (k, 0)),
        out_shape=jax.ShapeDtypeStruct((_NT * _TR, _H), jnp.float32),
    )(pe_table, w, b.reshape(1, _H))


_JPB = _L // _JB
_STEPS = ((_B * _L) // _NW) * _JPB


def _sc_fuse(tcat, pos_sf, pos_ef, asp, esp, out,
             pos_sv, pos_ev, aspv, espv, ix0, ix1, g0, g1, ob0, ob1,
             gsem0, gsem1, osem0, osem1):
    wid = lax.axis_index("s") * 2 + lax.axis_index("c")
    pltpu.sync_copy(pos_sf, pos_sv)
    pltpu.sync_copy(pos_ef, pos_ev)
    pltpu.sync_copy(asp, aspv)
    pltpu.sync_copy(esp, espv)
    rows_per_w = (_B * _L) // _NW
    r0 = wid * rows_per_w

    def build_idx(s, ix):
        r = r0 + s // _JPB
        jb = s % _JPB
        col0 = (r // _L) * _L + jb * _JB
        a_s = aspv[pl.ds(r * 16, 16)]
        a_e = espv[pl.ds(r * 16, 16)]
        for t in range(_JB // 16):
            sl = pl.ds(t * 16, 16)
            c_s = pos_sv[pl.ds(col0 + t * 16, 16)]
            c_e = pos_ev[pl.ds(col0 + t * 16, 16)]
            ix[0, sl] = a_s - c_s + _L
            ix[1, sl] = a_s - c_e + (_TR + _L)
            ix[2, sl] = a_e - c_s + (2 * _TR + _L)
            ix[3, sl] = a_e - c_e + (3 * _TR + _L)

    def out_slice(s):
        r = r0 + s // _JPB
        jb = s % _JPB
        return out.at[pl.ds(r * _L + jb * _JB, _JB)]

    zeros16 = jnp.zeros((16,), jnp.float32)

    def zero_buf(g):
        def zrow(j4, _):
            for u in range(4):
                j = j4 * 4 + u
                for h in range(_H // 16):
                    g[j, pl.ds(h * 16, 16)] = zeros16
            return 0

        lax.fori_loop(0, _JB // 4, zrow, 0)

    def fire_gathers(ix, g, gsem):
        for k in range(_NT):
            pltpu.async_copy(tcat.at[ix.at[k]], g, gsem, add=True)

    def drain_gathers(ix, g, gsem):
        for _ in range(_NT):
            pltpu.make_async_copy(tcat.at[ix.at[0]], g, gsem).wait()

    zero_buf(g0)
    zero_buf(g1)
    build_idx(0, ix0)
    fire_gathers(ix0, g0, gsem0)

    def step(s, _):
        par = s % 2

        def run(ix, g, gsem, osem, ob, oth_ix, oth_g, oth_gsem):
            @pl.when(s + 1 < _STEPS)
            def _prefetch():
                build_idx(s + 1, oth_ix)
                fire_gathers(oth_ix, oth_g, oth_gsem)

            drain_gathers(ix, g, gsem)

            @pl.when(s >= 2)
            def _free():
                pltpu.make_async_copy(ob, out_slice(s - 2), osem).wait()

            def jrow(j4, _):
                for u in range(4):
                    j = j4 * 4 + u
                    for h in range(_H // 16):
                        sl = pl.ds(h * 16, 16)
                        ob[j, sl] = jnp.maximum(g[j, sl], 0.0)
                        g[j, sl] = zeros16
                return 0

            lax.fori_loop(0, _JB // 4, jrow, 0)
            pltpu.async_copy(ob, out_slice(s), osem)

        @pl.when(par == 0)
        def _even():
            run(ix0, g0, gsem0, osem0, ob0, ix1, g1, gsem1)

        @pl.when(par == 1)
        def _odd():
            run(ix1, g1, gsem1, osem1, ob1, ix0, g0, gsem0)

        return 0

    lax.fori_loop(0, _STEPS, step, 0)
    pltpu.make_async_copy(ob0, out_slice(_STEPS - 2), osem0).wait()
    pltpu.make_async_copy(ob1, out_slice(_STEPS - 1), osem1).wait()


_sc_call = pl.kernel(
    _sc_fuse,
    mesh=plsc.VectorSubcoreMesh(core_axis_name="c", subcore_axis_name="s"),
    out_type=jax.ShapeDtypeStruct((_B * _L * _L, _H), jnp.float32),
    scratch_types=[
        pltpu.VMEM((_B * _L,), jnp.int32),
        pltpu.VMEM((_B * _L,), jnp.int32),
        pltpu.VMEM((_B * _L * 16,), jnp.int32),
        pltpu.VMEM((_B * _L * 16,), jnp.int32),
        pltpu.VMEM((_NT, _JB), jnp.int32),
        pltpu.VMEM((_NT, _JB), jnp.int32),
        pltpu.VMEM((_JB, _H), jnp.float32),
        pltpu.VMEM((_JB, _H), jnp.float32),
        pltpu.VMEM((_JB, _H), jnp.float32),
        pltpu.VMEM((_JB, _H), jnp.float32),
        pltpu.SemaphoreType.DMA,
        pltpu.SemaphoreType.DMA,
        pltpu.SemaphoreType.DMA,
        pltpu.SemaphoreType.DMA,
    ],
)


def kernel(pos_s, pos_e, pe_table, W, b):
    tcat = _build_tables(pe_table, W, b)
    psf = pos_s.reshape(-1).astype(jnp.int32)
    pef = pos_e.reshape(-1).astype(jnp.int32)
    asp = jnp.broadcast_to(psf[:, None], (_B * _L, 16)).reshape(-1)
    esp = jnp.broadcast_to(pef[:, None], (_B * _L, 16)).reshape(-1)
    out = _sc_call(tcat, psf, pef, asp, esp)
    return out.reshape(_B, _L, _L, _H)


# speedup vs baseline: 28.3148x; 1.3475x over previous
import functools

import jax
import jax.numpy as jnp
from jax import lax
from jax.experimental import pallas as pl
from jax.experimental.pallas import tpu as pltpu
from jax.experimental.pallas import tpu_sc as plsc

_B = 2
_L = 512
_H = 128
_NT = 4
_TR = 2 * _L
_JB = 128
_NW = 32


def _fuse_tables_body(pe_ref, w_ref, b_ref, out_ref):
    k = pl.program_id(0)
    acc = jnp.dot(pe_ref[...], w_ref[...], preferred_element_type=jnp.float32)
    out_ref[...] = acc + jnp.where(k == 0, b_ref[...], 0.0)


def _build_tables(pe_table, w, b):
    return pl.pallas_call(
        _fuse_tables_body,
        grid=(_NT,),
        in_specs=[
            pl.BlockSpec((_TR, _H), lambda k: (0, 0)),
            pl.BlockSpec((_H, _H), lambda k: (k, 0)),
            pl.BlockSpec((1, _H), lambda k: (0, 0)),
        ],
        out_specs=pl.BlockSpec((_TR, _H), lambda k: (k, 0)),
        out_shape=jax.ShapeDtypeStruct((_NT * _TR, _H), jnp.float32),
    )(pe_table, w, b.reshape(1, _H))


_JPB = _L // _JB
_STEPS = ((_B * _L) // _NW) * _JPB


def _sc_fuse(tcat, pos_sf, pos_ef, asp, esp, out,
             pos_sv, pos_ev, aspv, espv, ix0, ix1, g0, g1, ob0, ob1, tsh,
             gsem0, gsem1, osem0, osem1):
    wid = lax.axis_index("s") * 2 + lax.axis_index("c")

    @pl.when(lax.axis_index("s") == 0)
    def _stage():
        pltpu.sync_copy(tcat, tsh)

    rows_per_w = (_B * _L) // _NW
    r0 = wid * rows_per_w
    pltpu.sync_copy(pos_sf, pos_sv)
    pltpu.sync_copy(pos_ef, pos_ev)
    pltpu.sync_copy(asp.at[pl.ds(r0 * 16, rows_per_w * 16)], aspv)
    pltpu.sync_copy(esp.at[pl.ds(r0 * 16, rows_per_w * 16)], espv)
    plsc.subcore_barrier()

    def build_idx(s, ix):
        r = r0 + s // _JPB
        jb = s % _JPB
        col0 = (r // _L) * _L + jb * _JB
        a_s = aspv[pl.ds((r - r0) * 16, 16)]
        a_e = espv[pl.ds((r - r0) * 16, 16)]
        for t in range(_JB // 16):
            sl = pl.ds(t * 16, 16)
            c_s = pos_sv[pl.ds(col0 + t * 16, 16)]
            c_e = pos_ev[pl.ds(col0 + t * 16, 16)]
            ix[0, sl] = a_s - c_s + _L
            ix[1, sl] = a_s - c_e + (_TR + _L)
            ix[2, sl] = a_e - c_s + (2 * _TR + _L)
            ix[3, sl] = a_e - c_e + (3 * _TR + _L)

    def out_slice(s):
        r = r0 + s // _JPB
        jb = s % _JPB
        return out.at[pl.ds(r * _L + jb * _JB, _JB)]

    zeros16 = jnp.zeros((16,), jnp.float32)

    def zero_buf(g):
        def zrow(j4, _):
            for u in range(4):
                j = j4 * 4 + u
                for h in range(_H // 16):
                    g[j, pl.ds(h * 16, 16)] = zeros16
            return 0

        lax.fori_loop(0, _JB // 4, zrow, 0)

    def fire_gathers(ix, g, gsem):
        for k in range(_NT):
            pltpu.async_copy(tsh.at[ix.at[k]], g, gsem, add=True)

    def drain_gathers(ix, g, gsem):
        for _ in range(_NT):
            pltpu.make_async_copy(tsh.at[ix.at[0]], g, gsem).wait()

    zero_buf(g0)
    zero_buf(g1)
    build_idx(0, ix0)
    fire_gathers(ix0, g0, gsem0)

    def step(s, _):
        par = s % 2

        def run(ix, g, gsem, osem, ob, oth_ix, oth_g, oth_gsem):
            @pl.when(s + 1 < _STEPS)
            def _prefetch():
                build_idx(s + 1, oth_ix)
                fire_gathers(oth_ix, oth_g, oth_gsem)

            drain_gathers(ix, g, gsem)

            @pl.when(s >= 2)
            def _free():
                pltpu.make_async_copy(ob, out_slice(s - 2), osem).wait()

            def jrow(j4, _):
                for u in range(4):
                    j = j4 * 4 + u
                    for h in range(_H // 16):
                        sl = pl.ds(h * 16, 16)
                        ob[j, sl] = jnp.maximum(g[j, sl], 0.0)
                        g[j, sl] = zeros16
                return 0

            lax.fori_loop(0, _JB // 4, jrow, 0)
            pltpu.async_copy(ob, out_slice(s), osem)

        @pl.when(par == 0)
        def _even():
            run(ix0, g0, gsem0, osem0, ob0, ix1, g1, gsem1)

        @pl.when(par == 1)
        def _odd():
            run(ix1, g1, gsem1, osem1, ob1, ix0, g0, gsem0)

        return 0

    lax.fori_loop(0, _STEPS, step, 0)
    pltpu.make_async_copy(ob0, out_slice(_STEPS - 2), osem0).wait()
    pltpu.make_async_copy(ob1, out_slice(_STEPS - 1), osem1).wait()


_sc_call = pl.kernel(
    _sc_fuse,
    mesh=plsc.VectorSubcoreMesh(core_axis_name="c", subcore_axis_name="s"),
    out_type=jax.ShapeDtypeStruct((_B * _L * _L, _H), jnp.float32),
    scratch_types=[
        pltpu.VMEM((_B * _L,), jnp.int32),
        pltpu.VMEM((_B * _L,), jnp.int32),
        pltpu.VMEM(((_B * _L // _NW) * 16,), jnp.int32),
        pltpu.VMEM(((_B * _L // _NW) * 16,), jnp.int32),
        pltpu.VMEM((_NT, _JB), jnp.int32),
        pltpu.VMEM((_NT, _JB), jnp.int32),
        pltpu.VMEM((_JB, _H), jnp.float32),
        pltpu.VMEM((_JB, _H), jnp.float32),
        pltpu.VMEM((_JB, _H), jnp.float32),
        pltpu.VMEM((_JB, _H), jnp.float32),
        pltpu.VMEM_SHARED((_NT * _TR, _H), jnp.float32),
        pltpu.SemaphoreType.DMA,
        pltpu.SemaphoreType.DMA,
        pltpu.SemaphoreType.DMA,
        pltpu.SemaphoreType.DMA,
    ],
)


def kernel(pos_s, pos_e, pe_table, W, b):
    tcat = _build_tables(pe_table, W, b)
    psf = pos_s.reshape(-1).astype(jnp.int32)
    pef = pos_e.reshape(-1).astype(jnp.int32)
    asp = jnp.broadcast_to(psf[:, None], (_B * _L, 16)).reshape(-1)
    esp = jnp.broadcast_to(pef[:, None], (_B * _L, 16)).reshape(-1)
    out = _sc_call(tcat, psf, pef, asp, esp)
    return out.reshape(_B, _L, _L, _H)
